# Initial kernel scaffold; baseline (speedup 1.0000x reference)
import jax
import jax.numpy as jnp
from jax.experimental import pallas as pl


def kernel(col1, op, col2_or_num, is_join, col_emb, op_emb):
    raise NotImplementedError("write your pallas kernel here")


# speedup vs baseline: 1.7103x; 1.7103x over previous
import jax
import jax.numpy as jnp
from jax import lax
from jax.experimental import pallas as pl
from jax.experimental.pallas import tpu as pltpu
from jax.experimental.pallas import tpu_sc as plsc

B = 16384
L = 20
N = B * L
OUT_W = 138

_info = plsc.get_sparse_core_info()
NC = _info.num_cores
NS = _info.num_subcores
NW = NC * NS
RPW = N // NW
C = 256
CHUNKS = RPW // C


def _sc_body(col1_h, op_h, c2_h, join_h, tab_h, ope_h, out_h,
             idx1_v, opi_v, idx2_v, join_v, rows1_v, rows2_v, opebuf_v,
             tail_v, sem1, sem2, sem3):
    wid = lax.axis_index("s") * NC + lax.axis_index("c")
    wbase = wid * RPW
    pltpu.sync_copy(col1_h.at[pl.ds(wbase, RPW)], idx1_v)
    pltpu.sync_copy(op_h.at[pl.ds(wbase, RPW)], opi_v)
    pltpu.sync_copy(c2_h.at[pl.ds(wbase, RPW)], idx2_v)
    pltpu.sync_copy(join_h.at[pl.ds(wbase, RPW)], join_v)

    lanes = jnp.arange(16, dtype=jnp.int32)

    def chunk_body(ci, carry):
        cbase = pl.multiple_of(ci * C, C)
        obase = wbase + cbase
        cp1 = pltpu.async_copy(tab_h.at[idx1_v.at[pl.ds(cbase, C)]], rows1_v, sem1)
        cp2 = pltpu.async_copy(tab_h.at[idx2_v.at[pl.ds(cbase, C)]], rows2_v, sem2)
        cp3 = pltpu.async_copy(ope_h.at[opi_v.at[pl.ds(cbase, C)]], opebuf_v, sem3)

        cp1.wait()
        pltpu.sync_copy(rows1_v, out_h.at[pl.ds(obase, C), pl.ds(0, 64)])
        cp3.wait()
        pltpu.sync_copy(opebuf_v, out_h.at[pl.ds(obase, C), pl.ds(64, 16)])
        cp2.wait()

        def row_body(r, _):
            rsplat = jnp.zeros((16,), jnp.int32) + (cbase + r)
            g = plsc.load_gather(join_v, [rsplat]).astype(jnp.float32)
            for k in range(4):
                sl = pl.ds(k * 16, 16)
                rows2_v[r, sl] = rows2_v[r, sl] * g
            return 0

        lax.fori_loop(0, C, row_body, 0, unroll=4)
        pltpu.sync_copy(rows2_v, out_h.at[pl.ds(obase, C), pl.ds(72, 64)])

        def grp_body(g, _):
            off = cbase + g * 16
            jf = join_v[pl.ds(off, 16)].astype(jnp.float32)
            nf = idx2_v[pl.ds(off, 16)].astype(jnp.float32) * (1.0 - jf)
            ridx = g * 16 + lanes
            plsc.store_scatter(tail_v, [ridx, jnp.zeros((16,), jnp.int32)], nf)
            plsc.store_scatter(tail_v, [ridx, jnp.zeros((16,), jnp.int32) + 1], jf)
            return 0

        lax.fori_loop(0, C // 16, grp_body, 0, unroll=2)
        pltpu.sync_copy(tail_v, out_h.at[pl.ds(obase, C), pl.ds(136, 2)])
        return carry

    lax.fori_loop(0, CHUNKS, chunk_body, 0)


def kernel(col1, op, col2_or_num, is_join, col_emb, op_emb):
    col1f = col1.reshape(N).astype(jnp.int32)
    opf = op.reshape(N).astype(jnp.int32)
    c2f = col2_or_num.reshape(N).astype(jnp.int32)
    joinf = is_join.reshape(N).astype(jnp.int32)
    ope_p = jnp.pad(op_emb.astype(jnp.float32), ((0, 0), (0, 8)))
    mesh = plsc.VectorSubcoreMesh(core_axis_name="c", subcore_axis_name="s")
    out = pl.kernel(
        _sc_body,
        out_type=jax.ShapeDtypeStruct((N, OUT_W), jnp.float32),
        mesh=mesh,
        compiler_params=pltpu.CompilerParams(use_tc_tiling_on_sc=False,
                                             needs_layout_passes=False),
        scratch_types=[
            pltpu.VMEM((RPW,), jnp.int32),
            pltpu.VMEM((RPW,), jnp.int32),
            pltpu.VMEM((RPW,), jnp.int32),
            pltpu.VMEM((RPW,), jnp.int32),
            pltpu.VMEM((C, 64), jnp.float32),
            pltpu.VMEM((C, 64), jnp.float32),
            pltpu.VMEM((C, 16), jnp.float32),
            pltpu.VMEM((C, 2), jnp.float32),
            pltpu.SemaphoreType.DMA,
            pltpu.SemaphoreType.DMA,
            pltpu.SemaphoreType.DMA,
        ],
    )(col1f, opf, c2f, joinf, col_emb.astype(jnp.float32), ope_p)
    return out.reshape(B, L, OUT_W)


# speedup vs baseline: 3.4695x; 2.0286x over previous
import jax
import jax.numpy as jnp
from jax import lax
from jax.experimental import pallas as pl
from jax.experimental.pallas import tpu as pltpu
from jax.experimental.pallas import tpu_sc as plsc

B = 16384
L = 20
N = B * L
OUT_W = 138

_info = plsc.get_sparse_core_info()
NC = _info.num_cores
NS = _info.num_subcores
NW = NC * NS
RPW = N // NW
C = 160
CHUNKS = RPW // C


def _sc_body(col1_h, op_h, c2_h, join_h, tab_h, ope_h, out_h,
             idx1_v, opi_v, idx2_v, join_v, opetab_v,
             r1a, r2a, rga, r1b, r2b, rgb,
             sgi, s1a, s2a, soa, s1b, s2b, sob):
    wid = lax.axis_index("s") * NC + lax.axis_index("c")
    wbase = wid * RPW

    cps = [pltpu.async_copy(src.at[pl.ds(wbase, RPW)], dst, sgi)
           for src, dst in ((col1_h, idx1_v), (op_h, opi_v),
                            (c2_h, idx2_v), (join_h, join_v))]
    pltpu.sync_copy(ope_h, opetab_v)
    for cp in cps:
        cp.wait()

    lanes = jnp.arange(16, dtype=jnp.int32)
    zero16 = jnp.zeros((16,), jnp.int32)
    gcol = [lanes + (8 + 16 * k) for k in range(4)]
    m8 = lanes < 8
    c72 = zero16 + 72
    c73 = zero16 + 73

    def issue_gathers(ci, r1, r2, s1, s2):
        cb = pl.multiple_of(ci * C, 8)
        pltpu.async_copy(tab_h.at[idx1_v.at[pl.ds(cb, C)]], r1, s1)
        pltpu.async_copy(tab_h.at[idx2_v.at[pl.ds(cb, C)]], r2, s2)

    def wait_gathers(r1, r2, s1, s2):
        pltpu.make_async_copy(tab_h.at[idx1_v.at[pl.ds(0, C)]], r1, s1).wait()
        pltpu.make_async_copy(tab_h.at[idx2_v.at[pl.ds(0, C)]], r2, s2).wait()

    def compute(ci, r2, rg):
        cb = pl.multiple_of(ci * C, 8)

        def row_body(r, _):
            rsp = zero16 + r
            gidx = zero16 + (cb + r)
            g = plsc.load_gather(join_v, [gidx]).astype(jnp.float32)
            opid = plsc.load_gather(opi_v, [gidx])
            ov = plsc.load_gather(opetab_v, [opid, lanes])
            plsc.store_scatter(rg, [rsp, lanes], ov, mask=m8)
            for k in range(4):
                seg = r2[r, pl.ds(16 * k, 16)] * g
                plsc.store_scatter(rg, [rsp, gcol[k]], seg)
            return 0

        lax.fori_loop(0, C, row_body, 0, unroll=8)

        def grp_body(gg, _):
            loc = gg * 16
            rows_l = zero16 + loc + lanes
            jf = join_v[pl.ds(cb + loc, 16)].astype(jnp.float32)
            nf = idx2_v[pl.ds(cb + loc, 16)].astype(jnp.float32) * (1.0 - jf)
            plsc.store_scatter(rg, [rows_l, c72], nf)
            plsc.store_scatter(rg, [rows_l, c73], jf)
            return 0

        lax.fori_loop(0, C // 16, grp_body, 0, unroll=2)

    def issue_out(ci, r1, rg, so):
        ob = wbase + ci * C
        pltpu.async_copy(r1, out_h.at[pl.ds(ob, C), pl.ds(0, 64)], so)
        pltpu.async_copy(rg, out_h.at[pl.ds(ob, C), pl.ds(64, 74)], so)

    def wait_out(r1, rg, so):
        pltpu.make_async_copy(r1, out_h.at[pl.ds(0, C), pl.ds(0, 64)], so).wait()
        pltpu.make_async_copy(rg, out_h.at[pl.ds(0, C), pl.ds(64, 74)], so).wait()

    issue_gathers(0, r1a, r2a, s1a, s2a)
    issue_gathers(1, r1b, r2b, s1b, s2b)

    def pair_body(g, carry):
        a = 2 * g
        wait_gathers(r1a, r2a, s1a, s2a)
        compute(a, r2a, rga)
        issue_out(a, r1a, rga, soa)
        wait_gathers(r1b, r2b, s1b, s2b)
        compute(a + 1, r2b, rgb)
        issue_out(a + 1, r1b, rgb, sob)
        wait_out(r1a, rga, soa)

        @pl.when(a + 2 < CHUNKS)
        def _():
            issue_gathers(a + 2, r1a, r2a, s1a, s2a)

        wait_out(r1b, rgb, sob)

        @pl.when(a + 3 < CHUNKS)
        def _():
            issue_gathers(a + 3, r1b, r2b, s1b, s2b)

        return carry

    lax.fori_loop(0, CHUNKS // 2, pair_body, 0)


def kernel(col1, op, col2_or_num, is_join, col_emb, op_emb):
    col1f = col1.reshape(N).astype(jnp.int32)
    opf = op.reshape(N).astype(jnp.int32)
    c2f = col2_or_num.reshape(N).astype(jnp.int32)
    joinf = is_join.reshape(N).astype(jnp.int32)
    ope_p = jnp.pad(op_emb.astype(jnp.float32), ((0, 0), (0, 9)))
    mesh = plsc.VectorSubcoreMesh(core_axis_name="c", subcore_axis_name="s")
    out = pl.kernel(
        _sc_body,
        out_type=jax.ShapeDtypeStruct((N, OUT_W), jnp.float32),
        mesh=mesh,
        compiler_params=pltpu.CompilerParams(use_tc_tiling_on_sc=False,
                                             needs_layout_passes=False),
        scratch_types=[
            pltpu.VMEM((RPW,), jnp.int32),
            pltpu.VMEM((RPW,), jnp.int32),
            pltpu.VMEM((RPW,), jnp.int32),
            pltpu.VMEM((RPW,), jnp.int32),
            pltpu.VMEM((6, 17), jnp.float32),
            pltpu.VMEM((C, 64), jnp.float32),
            pltpu.VMEM((C, 64), jnp.float32),
            pltpu.VMEM((C, 74), jnp.float32),
            pltpu.VMEM((C, 64), jnp.float32),
            pltpu.VMEM((C, 64), jnp.float32),
            pltpu.VMEM((C, 74), jnp.float32),
            pltpu.SemaphoreType.DMA,
            pltpu.SemaphoreType.DMA,
            pltpu.SemaphoreType.DMA,
            pltpu.SemaphoreType.DMA,
            pltpu.SemaphoreType.DMA,
            pltpu.SemaphoreType.DMA,
            pltpu.SemaphoreType.DMA,
        ],
    )(col1f, opf, c2f, joinf, col_emb.astype(jnp.float32), ope_p)
    return out.reshape(B, L, OUT_W)


# speedup vs baseline: 5.9920x; 1.7271x over previous
import jax
import jax.numpy as jnp
from jax import lax
from jax.experimental import pallas as pl
from jax.experimental.pallas import tpu as pltpu
from jax.experimental.pallas import tpu_sc as plsc

B = 16384
L = 20
N = B * L
OUT_W = 138

_info = plsc.get_sparse_core_info()
NC = _info.num_cores
NS = _info.num_subcores
NW = NC * NS
RPW = N // NW
C = 40
CB = C // L
CHUNKS = RPW // C
NSLOT = 4


def _sc_body(col1_h, op_h, c2_h, join_h, tab_h, ope_h, out_h,
             idx1_v, opi_v, idx2_v, join_v, opetab_v,
             r1s, r2s, rgs, s1s, s2s, sos, sgi):
    wid = lax.axis_index("s") * NC + lax.axis_index("c")
    wbase = wid * RPW
    wb_b = wid * (RPW // L)

    cps = [pltpu.async_copy(src.at[pl.ds(wbase, RPW)], dst, sgi)
           for src, dst in ((col1_h, idx1_v), (op_h, opi_v),
                            (c2_h, idx2_v), (join_h, join_v))]
    pltpu.sync_copy(ope_h, opetab_v)
    for cp in cps:
        cp.wait()

    lanes = jnp.arange(16, dtype=jnp.int32)
    zero16 = jnp.zeros((16,), jnp.int32)
    one16 = zero16 + 1
    m8 = lanes < 8
    seg_ct = []
    seg_ci = []
    for cstart in (0, 16, 32, 48, 64, 72, 88, 104, 120):
        cv = lanes + cstart
        seg_ct.append(cv >> 7)
        seg_ci.append(cv & 127)
    ct136 = one16
    ci136 = zero16 + 8
    ci137 = zero16 + 9

    def issue_gathers(ci, r1, r2, s1, s2):
        cb = pl.multiple_of(ci * C, 8)
        pltpu.async_copy(tab_h.at[idx1_v.at[pl.ds(cb, C)]], r1, s1)
        pltpu.async_copy(tab_h.at[idx2_v.at[pl.ds(cb, C)]], r2, s2)

    def wait_gathers(r1, r2, s1, s2):
        pltpu.make_async_copy(tab_h.at[idx1_v.at[pl.ds(0, C)]], r1, s1).wait()
        pltpu.make_async_copy(tab_h.at[idx2_v.at[pl.ds(0, C)]], r2, s2).wait()

    def compute(ci, r1, r2, rg):
        cb = pl.multiple_of(ci * C, 8)

        def row_body(r, _):
            jb = r // L
            l = r % L
            lt = l // 8
            li = l % 8
            jbv = zero16 + jb
            ltv = zero16 + lt
            liv = zero16 + li
            gidx = zero16 + (cb + r)
            g = plsc.load_gather(join_v, [gidx]).astype(jnp.float32)
            opid = plsc.load_gather(opi_v, [gidx])
            ov = plsc.load_gather(opetab_v, [opid, lanes])
            for k in range(4):
                plsc.store_scatter(rg, [jbv, ltv, seg_ct[k], liv, seg_ci[k]],
                                   r1[r, pl.ds(16 * k, 16)])
            plsc.store_scatter(rg, [jbv, ltv, seg_ct[4], liv, seg_ci[4]],
                               ov, mask=m8)
            for k in range(4):
                seg = r2[r, pl.ds(16 * k, 16)] * g
                plsc.store_scatter(rg, [jbv, ltv, seg_ct[5 + k], liv, seg_ci[5 + k]],
                                   seg)
            return 0

        lax.fori_loop(0, C, row_body, 0, unroll=8)

        for gg in range((C + 15) // 16):
            loc = gg * 16
            nvalid = min(16, C - loc)
            msk = None if nvalid == 16 else (lanes < nvalid)
            rows_l = zero16 + loc + lanes
            jbv = rows_l // L
            lv = rows_l % L
            ltv = lv // 8
            liv = lv % 8
            jf = join_v[pl.ds(cb + loc, 16)].astype(jnp.float32)
            nf = idx2_v[pl.ds(cb + loc, 16)].astype(jnp.float32) * (1.0 - jf)
            plsc.store_scatter(rg, [jbv, ltv, ct136, liv, ci136], nf, mask=msk)
            plsc.store_scatter(rg, [jbv, ltv, ct136, liv, ci137], jf, mask=msk)

    def issue_out(ci, rg, so):
        b0 = wb_b + ci * CB
        for j in range(CB):
            pltpu.async_copy(rg.at[j, :, 0], out_h.at[b0 + j, :, 0], so)
            pltpu.async_copy(rg.at[j, :, 1, :, pl.ds(0, 16)],
                             out_h.at[b0 + j, :, 1, :, pl.ds(0, 16)], so)

    def wait_out(rg, so):
        for j in range(CB):
            pltpu.make_async_copy(rg.at[j, :, 0], out_h.at[0, :, 0], so).wait()
            pltpu.make_async_copy(rg.at[j, :, 1, :, pl.ds(0, 16)],
                                  out_h.at[0, :, 1, :, pl.ds(0, 16)], so).wait()

    issue_gathers(0, r1s[0], r2s[0], s1s[0], s2s[0])
    issue_gathers(1, r1s[1], r2s[1], s1s[1], s2s[1])

    def quad_body(q, carry):
        for s in range(NSLOT):
            c = NSLOT * q + s
            t = (s + 2) % NSLOT

            @pl.when(c >= 2)
            def _():
                wait_out(rgs[t], sos[t])

            @pl.when(c + 2 < CHUNKS)
            def _():
                issue_gathers(c + 2, r1s[t], r2s[t], s1s[t], s2s[t])

            wait_gathers(r1s[s], r2s[s], s1s[s], s2s[s])
            compute(c, r1s[s], r2s[s], rgs[s])
            issue_out(c, rgs[s], sos[s])
        return carry

    lax.fori_loop(0, CHUNKS // NSLOT, quad_body, 0)
    wait_out(rgs[2], sos[2])
    wait_out(rgs[3], sos[3])


def kernel(col1, op, col2_or_num, is_join, col_emb, op_emb):
    col1f = col1.reshape(N).astype(jnp.int32)
    opf = op.reshape(N).astype(jnp.int32)
    c2f = col2_or_num.reshape(N).astype(jnp.int32)
    joinf = is_join.reshape(N).astype(jnp.int32)
    ope_p = jnp.pad(op_emb.astype(jnp.float32), ((0, 0), (0, 9)))
    mesh = plsc.VectorSubcoreMesh(core_axis_name="c", subcore_axis_name="s")
    out = pl.kernel(
        _sc_body,
        out_type=jax.ShapeDtypeStruct((B, 3, 2, 8, 128), jnp.float32),
        mesh=mesh,
        compiler_params=pltpu.CompilerParams(use_tc_tiling_on_sc=False,
                                             needs_layout_passes=False),
        scratch_types=[
            pltpu.VMEM((RPW,), jnp.int32),
            pltpu.VMEM((RPW,), jnp.int32),
            pltpu.VMEM((RPW,), jnp.int32),
            pltpu.VMEM((RPW,), jnp.int32),
            pltpu.VMEM((6, 17), jnp.float32),
            [pltpu.VMEM((C, 64), jnp.float32) for _ in range(NSLOT)],
            [pltpu.VMEM((C, 64), jnp.float32) for _ in range(NSLOT)],
            [pltpu.VMEM((CB, 3, 2, 8, 128), jnp.float32) for _ in range(NSLOT)],
            [pltpu.SemaphoreType.DMA for _ in range(NSLOT)],
            [pltpu.SemaphoreType.DMA for _ in range(NSLOT)],
            [pltpu.SemaphoreType.DMA for _ in range(NSLOT)],
            pltpu.SemaphoreType.DMA,
        ],
    )(col1f, opf, c2f, joinf, col_emb.astype(jnp.float32), ope_p)
    y = out.transpose(0, 1, 3, 2, 4).reshape(B, 24, 256)
    return y[:, :L, :OUT_W]
